# baseline (speedup 1.0000x reference)
import functools

import jax
import jax.numpy as jnp
from jax import lax
from jax.experimental import pallas as pl
from jax.experimental.pallas import tpu as pltpu
from jax.experimental.pallas import tpu_sc as plsc

N = 10000
D = 128
H = 128
E0 = 320000

NC = 2
NS = 16
NW = NC * NS
NP = 10240
STRIPE = NP // NS
CH = 128
NCHUNK = 80
EPT = NCHUNK * CH
EP = EPT * NW


def _mesh():
    return plsc.VectorSubcoreMesh(
        core_axis_name="c", subcore_axis_name="s",
        num_cores=NC, num_subcores=NS)


@functools.cache
def _get_sc_counts():
    return functools.partial(
        pl.kernel,
        out_type=jax.ShapeDtypeStruct((NC * NP, H), jnp.float32),
        mesh=_mesh(),
        scratch_types=[
            pltpu.VMEM((CH,), jnp.int32),
            pltpu.VMEM((CH,), jnp.int32),
            pltpu.VMEM((CH,), jnp.int32),
            pltpu.VMEM((CH,), jnp.int32),
            pltpu.VMEM((CH, H), jnp.float32),
            pltpu.VMEM_SHARED((NP, H), jnp.float32),
            pltpu.SemaphoreType.DMA,
            pltpu.SemaphoreType.DMA,
            pltpu.SemaphoreType.DMA,
            pltpu.SemaphoreType.DMA,
        ],
    )(_sc_counts_body)


def _sc_counts_body(src_hbm, ones_hbm, zero_hbm, out_hbm,
                    idx0, idx1, idx2, idx3, ones_v, acc,
                    si0, si1, si2, si3):
    idx_c = (idx0, idx1, idx2, idx3)
    si = (si0, si1, si2, si3)
    cid = lax.axis_index("c")
    sid = lax.axis_index("s")
    wid = cid * NS + sid
    ebase = wid * EPT

    def _idx_issue(j, s):
        pltpu.async_copy(src_hbm.at[pl.ds(ebase + j * CH, CH)], idx_c[s], si[s])

    def _idx_wait(j, s):
        pltpu.make_async_copy(
            src_hbm.at[pl.ds(ebase + j * CH, CH)], idx_c[s], si[s]).wait()

    pltpu.sync_copy(ones_hbm, ones_v)
    pltpu.sync_copy(zero_hbm.at[pl.ds(sid * STRIPE, STRIPE)],
                    acc.at[pl.ds(sid * STRIPE, STRIPE)])
    plsc.subcore_barrier()

    for s in range(4):
        _idx_issue(s, s)
    NQ = NCHUNK // 4

    def _body(q, _):
        j0 = 4 * q
        more = q < NQ - 1
        for s in range(4):
            _idx_wait(j0 + s, s)
            pltpu.sync_copy(ones_v, acc.at[idx_c[s]], add=True)

            @pl.when(more)
            def _():
                _idx_issue(j0 + 4 + s, s)
        return 0
    lax.fori_loop(0, NQ, _body, 0)
    plsc.subcore_barrier()
    pltpu.sync_copy(acc.at[pl.ds(sid * STRIPE, STRIPE)],
                    out_hbm.at[pl.ds(cid * NP + sid * STRIPE, STRIPE)])


@functools.cache
def _get_sc_gather_scatter():
    return functools.partial(
        pl.kernel,
        out_type=jax.ShapeDtypeStruct((NC * NP, H), jnp.float32),
        mesh=_mesh(),
        scratch_types=(
            [pltpu.VMEM((CH,), jnp.int32) for _ in range(8)]
            + [pltpu.VMEM((CH, H), jnp.float32),
               pltpu.VMEM((CH, H), jnp.float32),
               pltpu.VMEM_SHARED((NP, H), jnp.float32)]
            + [pltpu.SemaphoreType.DMA for _ in range(8)]
        ),
    )(_sc_gs_body)


def _sc_gs_body(h_hbm, dst_hbm, src_hbm, zero_hbm, out_hbm,
                d0, d1, d2, d3, s0, s1, s2, s3,
                rows_a, rows_b, acc, sg_a, sg_b, ss_a, ss_b,
                si0, si1, si2, si3):
    dst_c = (d0, d1, d2, d3)
    src_c = (s0, s1, s2, s3)
    si = (si0, si1, si2, si3)
    cid = lax.axis_index("c")
    sid = lax.axis_index("s")
    wid = cid * NS + sid
    ebase = wid * EPT

    def _idx_issue(j, s):
        pltpu.async_copy(dst_hbm.at[pl.ds(ebase + j * CH, CH)], dst_c[s], si[s])
        pltpu.async_copy(src_hbm.at[pl.ds(ebase + j * CH, CH)], src_c[s], si[s])

    def _idx_wait(j, s):
        pltpu.make_async_copy(
            dst_hbm.at[pl.ds(ebase + j * CH, CH)], dst_c[s], si[s]).wait()
        pltpu.make_async_copy(
            src_hbm.at[pl.ds(ebase + j * CH, CH)], src_c[s], si[s]).wait()

    def _gather(s, rows, sg):
        pltpu.async_copy(h_hbm.at[dst_c[s]], rows, sg)

    def _gather_wait(s, rows, sg):
        pltpu.make_async_copy(h_hbm.at[dst_c[s]], rows, sg).wait()

    def _scatter(s, rows, ss):
        pltpu.async_copy(rows, acc.at[src_c[s]], ss, add=True)

    def _scatter_wait(s, rows, ss):
        pltpu.make_async_copy(rows, acc.at[src_c[s]], ss).wait()

    pltpu.sync_copy(zero_hbm.at[pl.ds(sid * STRIPE, STRIPE)],
                    acc.at[pl.ds(sid * STRIPE, STRIPE)])
    plsc.subcore_barrier()

    _idx_issue(0, 0)
    _idx_issue(1, 1)
    _idx_wait(0, 0)
    _gather(0, rows_a, sg_a)
    _idx_wait(1, 1)
    _gather(1, rows_b, sg_b)
    _idx_issue(2, 2)
    _idx_issue(3, 3)

    NQ = NCHUNK // 4

    def _body(q, _):
        j0 = 4 * q
        more = q < NQ - 1

        _gather_wait(0, rows_a, sg_a)
        _scatter(0, rows_a, ss_a)
        _gather_wait(1, rows_b, sg_b)
        _scatter(1, rows_b, ss_b)

        _scatter_wait(0, rows_a, ss_a)
        _idx_wait(j0 + 2, 2)
        _gather(2, rows_a, sg_a)

        @pl.when(more)
        def _():
            _idx_issue(j0 + 4, 0)

        _scatter_wait(1, rows_b, ss_b)
        _idx_wait(j0 + 3, 3)
        _gather(3, rows_b, sg_b)

        @pl.when(more)
        def _():
            _idx_issue(j0 + 5, 1)

        _gather_wait(2, rows_a, sg_a)
        _scatter(2, rows_a, ss_a)
        _gather_wait(3, rows_b, sg_b)
        _scatter(3, rows_b, ss_b)

        @pl.when(more)
        def _():
            _scatter_wait(2, rows_a, ss_a)
            _idx_wait(j0 + 4, 0)
            _gather(0, rows_a, sg_a)
            _idx_issue(j0 + 6, 2)
            _scatter_wait(3, rows_b, ss_b)
            _idx_wait(j0 + 5, 1)
            _gather(1, rows_b, sg_b)
            _idx_issue(j0 + 7, 3)
        return 0
    lax.fori_loop(0, NQ, _body, 0)
    _scatter_wait(2, rows_a, ss_a)
    _scatter_wait(3, rows_b, ss_b)
    plsc.subcore_barrier()
    pltpu.sync_copy(acc.at[pl.ds(sid * STRIPE, STRIPE)],
                    out_hbm.at[pl.ds(cid * NP + sid * STRIPE, STRIPE)])


_BR = 1024


def _mlp_body(x_ref, w1_ref, b1_ref, w2_ref, b2_ref, o_ref):
    t = jnp.dot(x_ref[...], w1_ref[...], preferred_element_type=jnp.float32)
    t = jnp.maximum(t + b1_ref[...], 0.0)
    o_ref[...] = (jnp.dot(t, w2_ref[...], preferred_element_type=jnp.float32)
                  + b2_ref[...])


def _tc_mlp(xp, w1, b1, w2, b2):
    return pl.pallas_call(
        _mlp_body,
        grid=(NP // _BR,),
        in_specs=[
            pl.BlockSpec((_BR, D), lambda i: (i, 0)),
            pl.BlockSpec((D, H), lambda i: (0, 0)),
            pl.BlockSpec((1, H), lambda i: (0, 0)),
            pl.BlockSpec((H, H), lambda i: (0, 0)),
            pl.BlockSpec((1, H), lambda i: (0, 0)),
        ],
        out_specs=pl.BlockSpec((_BR, H), lambda i: (i, 0)),
        out_shape=jax.ShapeDtypeStruct((NP, H), jnp.float32),
    )(xp, w1, b1.reshape(1, H), w2, b2.reshape(1, H))


def _comm_body(h_ref, p_ref, c_ref, w_ref, b_ref, o_ref):
    cnt = c_ref[0, :, 0:1] + c_ref[1, :, 0:1]
    sums = p_ref[0] + p_ref[1]
    msg = sums / jnp.maximum(cnt, 1.0)
    t = jnp.dot(msg, w_ref[...], preferred_element_type=jnp.float32)
    o_ref[...] = h_ref[...] + jnp.maximum(t + b_ref[...], 0.0)


def _tc_comm(h, p, c, w, b):
    return pl.pallas_call(
        _comm_body,
        grid=(NP // _BR,),
        in_specs=[
            pl.BlockSpec((_BR, H), lambda i: (i, 0)),
            pl.BlockSpec((NC, _BR, H), lambda i: (0, i, 0)),
            pl.BlockSpec((NC, _BR, H), lambda i: (0, i, 0)),
            pl.BlockSpec((H, H), lambda i: (0, 0)),
            pl.BlockSpec((1, H), lambda i: (0, 0)),
        ],
        out_specs=pl.BlockSpec((_BR, H), lambda i: (i, 0)),
        out_shape=jax.ShapeDtypeStruct((NP, H), jnp.float32),
    )(h, p, c, w, b.reshape(1, H))


def kernel(x, edge_index, enc_w1, enc_b1, enc_w2, enc_b2,
           comm_w0, comm_b0, comm_w1, comm_b1,
           dec_w1, dec_b1, dec_w2, dec_b2):
    xp = jnp.zeros((NP, D), jnp.float32).at[:N].set(x)
    src = edge_index[0]
    dst = edge_index[1]
    pad = EP - E0
    pad_src = N + (jnp.arange(pad, dtype=jnp.int32) % (NP - N))
    pad_dst = jnp.arange(pad, dtype=jnp.int32) % N
    srcp = jnp.concatenate([src, pad_src])
    dstp = jnp.concatenate([dst, pad_dst])
    onesH = jnp.ones((CH, H), jnp.float32)
    zeroH = jnp.zeros((NP, H), jnp.float32)

    c = _get_sc_counts()(srcp, onesH, zeroH).reshape(NC, NP, H)
    h = _tc_mlp(xp, enc_w1, enc_b1, enc_w2, enc_b2)
    for (w, b) in ((comm_w0, comm_b0), (comm_w1, comm_b1)):
        p = _get_sc_gather_scatter()(h, dstp, srcp, zeroH).reshape(NC, NP, H)
        h = _tc_comm(h, p, c, w, b)
    out = _tc_mlp(h, dec_w1, dec_b1, dec_w2, dec_b2)
    return out[:N]


# speedup vs baseline: 1.1962x; 1.1962x over previous
import functools

import jax
import jax.numpy as jnp
from jax import lax
from jax.experimental import pallas as pl
from jax.experimental.pallas import tpu as pltpu
from jax.experimental.pallas import tpu_sc as plsc

N = 10000
D = 128
H = 128
E0 = 320000

NC = 2
NS = 16
NW = NC * NS
NP = 10240
STRIPE = NP // NS
CH = 128
NCHUNK = 80
EPT = NCHUNK * CH
EP = EPT * NW


def _mesh():
    return plsc.VectorSubcoreMesh(
        core_axis_name="c", subcore_axis_name="s",
        num_cores=NC, num_subcores=NS)


@functools.cache
def _get_sc_counts():
    return functools.partial(
        pl.kernel,
        out_type=jax.ShapeDtypeStruct((NC * NP, H), jnp.float32),
        mesh=_mesh(),
        scratch_types=[
            pltpu.VMEM((CH,), jnp.int32),
            pltpu.VMEM((CH,), jnp.int32),
            pltpu.VMEM((CH,), jnp.int32),
            pltpu.VMEM((CH,), jnp.int32),
            pltpu.VMEM((CH, H), jnp.float32),
            pltpu.VMEM_SHARED((NP, H), jnp.float32),
            pltpu.SemaphoreType.DMA,
            pltpu.SemaphoreType.DMA,
            pltpu.SemaphoreType.DMA,
            pltpu.SemaphoreType.DMA,
        ],
    )(_sc_counts_body)


def _sc_counts_body(src_hbm, ones_hbm, zero_hbm, out_hbm,
                    idx0, idx1, idx2, idx3, ones_v, acc,
                    si0, si1, si2, si3):
    idx_c = (idx0, idx1, idx2, idx3)
    si = (si0, si1, si2, si3)
    cid = lax.axis_index("c")
    sid = lax.axis_index("s")
    wid = cid * NS + sid
    ebase = wid * EPT

    def _idx_issue(j, s):
        pltpu.async_copy(src_hbm.at[pl.ds(ebase + j * CH, CH)], idx_c[s], si[s])

    def _idx_wait(j, s):
        pltpu.make_async_copy(
            src_hbm.at[pl.ds(ebase + j * CH, CH)], idx_c[s], si[s]).wait()

    pltpu.sync_copy(ones_hbm, ones_v)
    pltpu.sync_copy(zero_hbm.at[pl.ds(sid * STRIPE, STRIPE)],
                    acc.at[pl.ds(sid * STRIPE, STRIPE)])
    plsc.subcore_barrier()

    for s in range(4):
        _idx_issue(s, s)
    NQ = NCHUNK // 4

    def _body(q, _):
        j0 = 4 * q
        more = q < NQ - 1
        for s in range(4):
            _idx_wait(j0 + s, s)
            pltpu.sync_copy(ones_v, acc.at[idx_c[s]], add=True)

            @pl.when(more)
            def _():
                _idx_issue(j0 + 4 + s, s)
        return 0
    lax.fori_loop(0, NQ, _body, 0)
    plsc.subcore_barrier()
    pltpu.sync_copy(acc.at[pl.ds(sid * STRIPE, STRIPE)],
                    out_hbm.at[pl.ds(cid * NP + sid * STRIPE, STRIPE)])


@functools.cache
def _get_sc_gather_scatter():
    return functools.partial(
        pl.kernel,
        out_type=jax.ShapeDtypeStruct((NC * NP, H), jnp.float32),
        mesh=_mesh(),
        scratch_types=(
            [pltpu.VMEM((CH,), jnp.int32) for _ in range(8)]
            + [pltpu.VMEM((CH, H), jnp.float32),
               pltpu.VMEM((CH, H), jnp.float32),
               pltpu.VMEM_SHARED((NP, H), jnp.float32)]
            + [pltpu.SemaphoreType.DMA for _ in range(6)]
        ),
    )(_sc_gs_body)


def _sc_gs_body(h_hbm, dst_hbm, src_hbm, zero_hbm, out_hbm,
                d0, d1, d2, d3, s0, s1, s2, s3,
                rows_a, rows_b, acc, sg_a, sg_b, si0, si1, si2, si3):
    dst_c = (d0, d1, d2, d3)
    src_c = (s0, s1, s2, s3)
    si = (si0, si1, si2, si3)
    cid = lax.axis_index("c")
    sid = lax.axis_index("s")
    wid = cid * NS + sid
    ebase = wid * EPT

    def _idx_issue(j, s):
        pltpu.async_copy(dst_hbm.at[pl.ds(ebase + j * CH, CH)], dst_c[s], si[s])
        pltpu.async_copy(src_hbm.at[pl.ds(ebase + j * CH, CH)], src_c[s], si[s])

    def _idx_wait(j, s):
        pltpu.make_async_copy(
            dst_hbm.at[pl.ds(ebase + j * CH, CH)], dst_c[s], si[s]).wait()
        pltpu.make_async_copy(
            src_hbm.at[pl.ds(ebase + j * CH, CH)], src_c[s], si[s]).wait()

    def _gather(s, rows, sg):
        pltpu.async_copy(h_hbm.at[dst_c[s]], rows, sg)

    def _scatter(s, rows, sg):
        pltpu.make_async_copy(h_hbm.at[dst_c[s]], rows, sg).wait()
        pltpu.sync_copy(rows, acc.at[src_c[s]], add=True)

    pltpu.sync_copy(zero_hbm.at[pl.ds(sid * STRIPE, STRIPE)],
                    acc.at[pl.ds(sid * STRIPE, STRIPE)])
    plsc.subcore_barrier()

    _idx_issue(0, 0)
    _idx_issue(1, 1)
    _idx_wait(0, 0)
    _gather(0, rows_a, sg_a)
    _idx_wait(1, 1)
    _gather(1, rows_b, sg_b)
    _idx_issue(2, 2)
    _idx_issue(3, 3)

    NQ = NCHUNK // 4

    def _body(q, _):
        j0 = 4 * q
        more = q < NQ - 1

        _scatter(0, rows_a, sg_a)
        _idx_wait(j0 + 2, 2)
        _gather(2, rows_a, sg_a)

        @pl.when(more)
        def _():
            _idx_issue(j0 + 4, 0)

        _scatter(1, rows_b, sg_b)
        _idx_wait(j0 + 3, 3)
        _gather(3, rows_b, sg_b)

        @pl.when(more)
        def _():
            _idx_issue(j0 + 5, 1)

        _scatter(2, rows_a, sg_a)

        @pl.when(more)
        def _():
            _idx_wait(j0 + 4, 0)
            _gather(0, rows_a, sg_a)
            _idx_issue(j0 + 6, 2)

        _scatter(3, rows_b, sg_b)

        @pl.when(more)
        def _():
            _idx_wait(j0 + 5, 1)
            _gather(1, rows_b, sg_b)
            _idx_issue(j0 + 7, 3)
        return 0
    lax.fori_loop(0, NQ, _body, 0)
    plsc.subcore_barrier()
    pltpu.sync_copy(acc.at[pl.ds(sid * STRIPE, STRIPE)],
                    out_hbm.at[pl.ds(cid * NP + sid * STRIPE, STRIPE)])


_BR = 1024


def _mlp_body(x_ref, w1_ref, b1_ref, w2_ref, b2_ref, o_ref):
    t = jnp.dot(x_ref[...], w1_ref[...], preferred_element_type=jnp.float32)
    t = jnp.maximum(t + b1_ref[...], 0.0)
    o_ref[...] = (jnp.dot(t, w2_ref[...], preferred_element_type=jnp.float32)
                  + b2_ref[...])


def _tc_mlp(xp, w1, b1, w2, b2):
    return pl.pallas_call(
        _mlp_body,
        grid=(NP // _BR,),
        in_specs=[
            pl.BlockSpec((_BR, D), lambda i: (i, 0)),
            pl.BlockSpec((D, H), lambda i: (0, 0)),
            pl.BlockSpec((1, H), lambda i: (0, 0)),
            pl.BlockSpec((H, H), lambda i: (0, 0)),
            pl.BlockSpec((1, H), lambda i: (0, 0)),
        ],
        out_specs=pl.BlockSpec((_BR, H), lambda i: (i, 0)),
        out_shape=jax.ShapeDtypeStruct((NP, H), jnp.float32),
    )(xp, w1, b1.reshape(1, H), w2, b2.reshape(1, H))


def _comm_body(h_ref, p_ref, c_ref, w_ref, b_ref, o_ref):
    cnt = c_ref[0, :, 0:1] + c_ref[1, :, 0:1]
    sums = p_ref[0] + p_ref[1]
    msg = sums / jnp.maximum(cnt, 1.0)
    t = jnp.dot(msg, w_ref[...], preferred_element_type=jnp.float32)
    o_ref[...] = h_ref[...] + jnp.maximum(t + b_ref[...], 0.0)


def _tc_comm(h, p, c, w, b):
    return pl.pallas_call(
        _comm_body,
        grid=(NP // _BR,),
        in_specs=[
            pl.BlockSpec((_BR, H), lambda i: (i, 0)),
            pl.BlockSpec((NC, _BR, H), lambda i: (0, i, 0)),
            pl.BlockSpec((NC, _BR, H), lambda i: (0, i, 0)),
            pl.BlockSpec((H, H), lambda i: (0, 0)),
            pl.BlockSpec((1, H), lambda i: (0, 0)),
        ],
        out_specs=pl.BlockSpec((_BR, H), lambda i: (i, 0)),
        out_shape=jax.ShapeDtypeStruct((NP, H), jnp.float32),
    )(h, p, c, w, b.reshape(1, H))


def kernel(x, edge_index, enc_w1, enc_b1, enc_w2, enc_b2,
           comm_w0, comm_b0, comm_w1, comm_b1,
           dec_w1, dec_b1, dec_w2, dec_b2):
    xp = jnp.zeros((NP, D), jnp.float32).at[:N].set(x)
    src = edge_index[0]
    dst = edge_index[1]
    pad = EP - E0
    pad_src = N + (jnp.arange(pad, dtype=jnp.int32) % (NP - N))
    pad_dst = jnp.arange(pad, dtype=jnp.int32) % N
    srcp = jnp.concatenate([src, pad_src])
    dstp = jnp.concatenate([dst, pad_dst])
    onesH = jnp.ones((CH, H), jnp.float32)
    zeroH = jnp.zeros((NP, H), jnp.float32)

    c = _get_sc_counts()(srcp, onesH, zeroH).reshape(NC, NP, H)
    h = _tc_mlp(xp, enc_w1, enc_b1, enc_w2, enc_b2)
    for (w, b) in ((comm_w0, comm_b0), (comm_w1, comm_b1)):
        p = _get_sc_gather_scatter()(h, dstp, srcp, zeroH).reshape(NC, NP, H)
        h = _tc_comm(h, p, c, w, b)
    out = _tc_mlp(h, dec_w1, dec_b1, dec_w2, dec_b2)
    return out[:N]
